# baseline (speedup 1.0000x reference)
import functools

import jax
import jax.numpy as jnp
import numpy as np
from jax.experimental import pallas as pl
from jax.experimental.pallas import tpu as pltpu

_NUM_TABLE = 32
_CODE_LENGTH = 16
_HIDDEN = 2048
_TOTAL = _NUM_TABLE * _CODE_LENGTH
_BM = 1024

_COLS = np.arange(_TOTAL)
_A, _REM = _COLS // 128, _COLS % 128
_Q, _T = _REM // 32, _REM % 32
_K = 4 * _A + _Q
_PMAT = np.zeros((_TOTAL, _TOTAL), np.float32)
_PMAT[_CODE_LENGTH * _T + _K, _COLS] = 1.0
_SELC = np.zeros((_TOTAL, _NUM_TABLE), np.float32)
_SELC[_COLS, _T] = 2.0 ** _K


def _fused_kernel(x_ref, w_ref, b_ref, p_ref, selc_ref, code_ref, score_ref):
    z = jnp.dot(x_ref[...].astype(jnp.bfloat16), w_ref[...].astype(jnp.bfloat16),
                preferred_element_type=jnp.float32)
    z = z + b_ref[...]
    zp = jax.lax.dot(z, p_ref[...], preferred_element_type=jnp.float32)
    bits = (zp > 0).astype(jnp.float32)
    codef = jax.lax.dot(bits, selc_ref[...],
                        preferred_element_type=jnp.float32)
    code_ref[...] = codef.astype(jnp.int32)
    za = jnp.abs(zp)
    m = (za[:, 0:128] * za[:, 128:256]) * (za[:, 256:384] * za[:, 384:512])
    m = m * pltpu.roll(m, 96, 1)
    m = m * pltpu.roll(m, 64, 1)
    score_ref[...] = m[:, 0:_NUM_TABLE]


@functools.partial(jax.jit, static_argnames=("interpret",))
def kernel(x, W, b, interpret=False):
    Bsz = x.shape[0]
    grid = (Bsz // _BM,)
    code, score = pl.pallas_call(
        _fused_kernel,
        grid=grid,
        in_specs=[
            pl.BlockSpec((_BM, _HIDDEN), lambda i: (i, 0)),
            pl.BlockSpec((_HIDDEN, _TOTAL), lambda i: (0, 0)),
            pl.BlockSpec((1, _TOTAL), lambda i: (0, 0)),
            pl.BlockSpec((_TOTAL, _TOTAL), lambda i: (0, 0)),
            pl.BlockSpec((_TOTAL, _NUM_TABLE), lambda i: (0, 0)),
        ],
        out_specs=[
            pl.BlockSpec((_BM, _NUM_TABLE), lambda i: (i, 0)),
            pl.BlockSpec((_BM, _NUM_TABLE), lambda i: (i, 0)),
        ],
        out_shape=[
            jax.ShapeDtypeStruct((Bsz, _NUM_TABLE), jnp.int32),
            jax.ShapeDtypeStruct((Bsz, _NUM_TABLE), jnp.float32),
        ],
        compiler_params=pltpu.CompilerParams(
            dimension_semantics=("parallel",)),
        interpret=interpret,
    )(x, W, b.reshape(1, _TOTAL), jnp.asarray(_PMAT), jnp.asarray(_SELC))
    return (code, score)


# speedup vs baseline: 1.0035x; 1.0035x over previous
import functools

import jax
import jax.numpy as jnp
import numpy as np
from jax.experimental import pallas as pl
from jax.experimental.pallas import tpu as pltpu

_NUM_TABLE = 32
_CODE_LENGTH = 16
_HIDDEN = 2048
_TOTAL = _NUM_TABLE * _CODE_LENGTH
_BM = 1024

_COLS = np.arange(_TOTAL)
_A, _REM = _COLS // 128, _COLS % 128
_Q, _T = _REM // 32, _REM % 32
_K = 4 * _A + _Q
_PMAT = np.zeros((_TOTAL, _TOTAL), np.float32)
_PMAT[_CODE_LENGTH * _T + _K, _COLS] = 1.0
_SELC = np.zeros((_TOTAL, _NUM_TABLE), np.float32)
_SELC[_COLS, _COLS // _CODE_LENGTH] = 2.0 ** (_COLS % _CODE_LENGTH)


def _fused_kernel(x_ref, w_ref, b_ref, p_ref, selc_ref, code_ref, score_ref):
    z = jnp.dot(x_ref[...], w_ref[...], preferred_element_type=jnp.float32)
    z = z + b_ref[...]
    bits = (z > 0).astype(jnp.float32)
    codef = jax.lax.dot(bits, selc_ref[...],
                        preferred_element_type=jnp.float32)
    code_ref[...] = codef.astype(jnp.int32)
    zp = jax.lax.dot(z, p_ref[...], preferred_element_type=jnp.float32)
    za = jnp.abs(zp)
    m = (za[:, 0:128] * za[:, 128:256]) * (za[:, 256:384] * za[:, 384:512])
    m = m * pltpu.roll(m, 96, 1)
    m = m * pltpu.roll(m, 64, 1)
    score_ref[...] = m[:, 0:_NUM_TABLE]


@functools.partial(jax.jit, static_argnames=("interpret",))
def kernel(x, W, b, interpret=False):
    Bsz = x.shape[0]
    grid = (Bsz // _BM,)
    code, score = pl.pallas_call(
        _fused_kernel,
        grid=grid,
        in_specs=[
            pl.BlockSpec((_BM, _HIDDEN), lambda i: (i, 0)),
            pl.BlockSpec((_HIDDEN, _TOTAL), lambda i: (0, 0)),
            pl.BlockSpec((1, _TOTAL), lambda i: (0, 0)),
            pl.BlockSpec((_TOTAL, _TOTAL), lambda i: (0, 0)),
            pl.BlockSpec((_TOTAL, _NUM_TABLE), lambda i: (0, 0)),
        ],
        out_specs=[
            pl.BlockSpec((_BM, _NUM_TABLE), lambda i: (i, 0)),
            pl.BlockSpec((_BM, _NUM_TABLE), lambda i: (i, 0)),
        ],
        out_shape=[
            jax.ShapeDtypeStruct((Bsz, _NUM_TABLE), jnp.int32),
            jax.ShapeDtypeStruct((Bsz, _NUM_TABLE), jnp.float32),
        ],
        compiler_params=pltpu.CompilerParams(
            dimension_semantics=("parallel",)),
        interpret=interpret,
    )(x, W, b.reshape(1, _TOTAL), jnp.asarray(_PMAT), jnp.asarray(_SELC))
    return (code, score)
